# baseline (speedup 1.0000x reference)
import jax
import jax.numpy as jnp
from jax.experimental import pallas as pl
from jax.experimental.pallas import tpu as pltpu

_BN = 4608


def _mlp_block(xt_ref, w1_ref, b1_ref, wo_ref, bo_ref, out_ref):
    h = jnp.dot(w1_ref[...], xt_ref[...], preferred_element_type=jnp.float32,
                precision=jax.lax.Precision.DEFAULT)
    h = jnp.maximum(h + b1_ref[...], 0.0)
    out = jnp.dot(wo_ref[...], h, preferred_element_type=jnp.float32,
                  precision=jax.lax.Precision.DEFAULT)
    out_ref[...] = out + bo_ref[...]


def kernel(x, W1, b1, W_out, b_out):
    n, d = x.shape
    hdim = W1.shape[0]
    c = W_out.shape[0]
    xt = x.T
    b1r = b1.reshape(hdim, 1)
    bor = b_out.reshape(c, 1)
    grid = (pl.cdiv(n, _BN),)
    outt = pl.pallas_call(
        _mlp_block,
        grid=grid,
        in_specs=[
            pl.BlockSpec((d, _BN), lambda j: (0, j)),
            pl.BlockSpec((hdim, d), lambda j: (0, 0)),
            pl.BlockSpec((hdim, 1), lambda j: (0, 0)),
            pl.BlockSpec((c, hdim), lambda j: (0, 0)),
            pl.BlockSpec((c, 1), lambda j: (0, 0)),
        ],
        out_specs=pl.BlockSpec((c, _BN), lambda j: (0, j)),
        out_shape=jax.ShapeDtypeStruct((c, n), jnp.float32),
        compiler_params=pltpu.CompilerParams(
            dimension_semantics=("arbitrary",)),
    )(xt, W1, b1r, W_out, bor)
    return outt.T


# speedup vs baseline: 1.0140x; 1.0140x over previous
import jax
import jax.numpy as jnp
from jax.experimental import pallas as pl
from jax.experimental.pallas import tpu as pltpu

_BN = 5632


def _mlp_block(xt_ref, w1_ref, b1_ref, wo_ref, bo_ref, out_ref):
    h = jnp.dot(w1_ref[...], xt_ref[...], preferred_element_type=jnp.float32,
                precision=jax.lax.Precision.DEFAULT)
    h = jnp.maximum(h + b1_ref[...], 0.0)
    out = jnp.dot(wo_ref[...], h, preferred_element_type=jnp.float32,
                  precision=jax.lax.Precision.DEFAULT)
    out_ref[...] = out + bo_ref[...]


def kernel(x, W1, b1, W_out, b_out):
    n, d = x.shape
    hdim = W1.shape[0]
    c = W_out.shape[0]
    xt = x.T
    b1r = b1.reshape(hdim, 1)
    bor = b_out.reshape(c, 1)
    grid = (pl.cdiv(n, _BN),)
    outt = pl.pallas_call(
        _mlp_block,
        grid=grid,
        in_specs=[
            pl.BlockSpec((d, _BN), lambda j: (0, j)),
            pl.BlockSpec((hdim, d), lambda j: (0, 0)),
            pl.BlockSpec((hdim, 1), lambda j: (0, 0)),
            pl.BlockSpec((c, hdim), lambda j: (0, 0)),
            pl.BlockSpec((c, 1), lambda j: (0, 0)),
        ],
        out_specs=pl.BlockSpec((c, _BN), lambda j: (0, j)),
        out_shape=jax.ShapeDtypeStruct((c, n), jnp.float32),
        compiler_params=pltpu.CompilerParams(
            dimension_semantics=("arbitrary",)),
    )(xt, W1, b1r, W_out, bor)
    return outt.T


# speedup vs baseline: 1.0184x; 1.0043x over previous
import jax
import jax.numpy as jnp
from jax.experimental import pallas as pl
from jax.experimental.pallas import tpu as pltpu

_BN = 5120


def _mlp_block(xt_ref, w1_ref, b1_ref, wo_ref, bo_ref, out_ref):
    h = jnp.dot(w1_ref[...], xt_ref[...], preferred_element_type=jnp.float32,
                precision=jax.lax.Precision.DEFAULT)
    h = jnp.maximum(h + b1_ref[...], 0.0)
    out = jnp.dot(wo_ref[...], h, preferred_element_type=jnp.float32,
                  precision=jax.lax.Precision.DEFAULT)
    out_ref[...] = out + bo_ref[...]


def kernel(x, W1, b1, W_out, b_out):
    n, d = x.shape
    hdim = W1.shape[0]
    c = W_out.shape[0]
    xt = x.T
    b1r = b1.reshape(hdim, 1)
    bor = b_out.reshape(c, 1)
    grid = (pl.cdiv(n, _BN),)
    outt = pl.pallas_call(
        _mlp_block,
        grid=grid,
        in_specs=[
            pl.BlockSpec((d, _BN), lambda j: (0, j)),
            pl.BlockSpec((hdim, d), lambda j: (0, 0)),
            pl.BlockSpec((hdim, 1), lambda j: (0, 0)),
            pl.BlockSpec((c, hdim), lambda j: (0, 0)),
            pl.BlockSpec((c, 1), lambda j: (0, 0)),
        ],
        out_specs=pl.BlockSpec((c, _BN), lambda j: (0, j)),
        out_shape=jax.ShapeDtypeStruct((c, n), jnp.float32),
        compiler_params=pltpu.CompilerParams(
            dimension_semantics=("arbitrary",)),
    )(xt, W1, b1r, W_out, bor)
    return outt.T
